# baseline (speedup 1.0000x reference)
import functools

import jax
import jax.numpy as jnp
from jax import lax
from jax.experimental import pallas as pl
from jax.experimental.pallas import tpu as pltpu
from jax.experimental.pallas import tpu_sc as plsc

N = 50000
NPAD = 50048
F = 16
NC = 2
NS = 16
B = 1000
BPAD = 1008
Q = 40
SUBS = B // Q
ROWS_PER_TILE = NPAD // NS


def _sc_segment_accumulate(wcoord, idx2d, weights):
    E = weights.shape[0]
    EP = E // NS
    NCHUNK = EP // B

    mesh = plsc.VectorSubcoreMesh(core_axis_name="c", subcore_axis_name="s")

    @functools.partial(
        pl.kernel,
        mesh=mesh,
        compiler_params=pltpu.CompilerParams(use_tc_tiling_on_sc=False),
        out_type=(
            jax.ShapeDtypeStruct((NPAD, F), jnp.float32),
            jax.ShapeDtypeStruct((NPAD, F), jnp.float32),
        ),
        scratch_types=[
            pltpu.VMEM((SUBS, Q), jnp.int32),
            pltpu.VMEM((BPAD,), jnp.float32),
            pltpu.VMEM((BPAD, F), jnp.float32),
            pltpu.VMEM_SHARED((NPAD, F), jnp.float32),
            pltpu.SemaphoreType.DMA,
            pltpu.SemaphoreType.DMA,
        ],
    )
    def sc_kernel(wc_hbm, idx_hbm, w_hbm, outf_hbm, outw_hbm,
                  idx_v, w_v, pay_v, acc, sem_in, sem_sc):
        c = lax.axis_index("c")
        s = lax.axis_index("s")

        def zrow(i, carry):
            pay_v[i, pl.ds(0, F)] = jnp.zeros((16,), jnp.float32)
            return carry
        lax.fori_loop(0, BPAD, zrow, 0)
        r0 = s * ROWS_PER_TILE
        for off, sz in ((0, 1000), (1000, 1000), (2000, 1000), (3000, 128)):
            pltpu.sync_copy(pay_v.at[pl.ds(0, sz), :],
                            acc.at[pl.ds(r0 + off, sz), :])
        plsc.subcore_barrier()

        @pl.when(c == 0)
        def _feature_role():
            def chunk_body(ci, carry):
                base = s * EP + ci * B
                cp1 = pltpu.async_copy(idx_hbm.at[pl.ds(base // Q, SUBS), :],
                                       idx_v, sem_in)
                cp2 = pltpu.async_copy(wc_hbm.at[pl.ds(base, B), :],
                                       pay_v.at[pl.ds(0, B), :], sem_in)
                cp1.wait()
                cp2.wait()
                scs = []
                for j in range(SUBS):
                    scs.append(pltpu.async_copy(
                        pay_v.at[pl.ds(j * Q, Q), :],
                        acc.at[idx_v.at[j]], sem_sc, add=True))
                for d in scs:
                    d.wait()
                return carry
            lax.fori_loop(0, NCHUNK, chunk_body, 0)

        @pl.when(c == 1)
        def _weight_role():
            def chunk_body(ci, carry):
                base = s * EP + ci * B
                cp1 = pltpu.async_copy(idx_hbm.at[pl.ds(base // Q, SUBS), :],
                                       idx_v, sem_in)
                cp2 = pltpu.async_copy(w_hbm.at[pl.ds(base, B)],
                                       w_v.at[pl.ds(0, B)], sem_in)
                cp1.wait()
                cp2.wait()

                def g_body(g, carry2):
                    e0 = g * 16
                    wvec = w_v[pl.ds(e0, 16)]
                    for j in range(16):
                        pay_v[e0 + j, pl.ds(0, F)] = jnp.full(
                            (16,), wvec[j], jnp.float32)
                    return carry2
                lax.fori_loop(0, BPAD // 16, g_body, 0)
                scs = []
                for j in range(SUBS):
                    scs.append(pltpu.async_copy(
                        pay_v.at[pl.ds(j * Q, Q), :],
                        acc.at[idx_v.at[j]], sem_sc, add=True))
                for d in scs:
                    d.wait()
                return carry
            lax.fori_loop(0, NCHUNK, chunk_body, 0)

        plsc.subcore_barrier()
        @pl.when(c == 0)
        def _pub_f():
            pltpu.sync_copy(acc.at[pl.ds(r0, ROWS_PER_TILE), :],
                            outf_hbm.at[pl.ds(r0, ROWS_PER_TILE), :])

        @pl.when(c == 1)
        def _pub_w():
            pltpu.sync_copy(acc.at[pl.ds(r0, ROWS_PER_TILE), :],
                            outw_hbm.at[pl.ds(r0, ROWS_PER_TILE), :])

    return sc_kernel(wcoord, idx2d, weights)


def _tc_finish_body(pf_ref, pw_ref, emb_ref, o_ref):
    feat = pf_ref[...]
    ws = pw_ref[:, 0:1]
    y = lax.dot_general(feat, emb_ref[...],
                        (((1,), (1,)), ((), ())),
                        preferred_element_type=jnp.float32)
    o_ref[...] = y / ws


def _tc_finish(pf, pw, embedding):
    U = embedding.shape[0]
    R = 2000
    grid = (N // R,)
    return pl.pallas_call(
        _tc_finish_body,
        grid=grid,
        in_specs=[
            pl.BlockSpec((R, F), lambda i: (i, 0)),
            pl.BlockSpec((R, F), lambda i: (i, 0)),
            pl.BlockSpec((U, F), lambda i: (0, 0)),
        ],
        out_specs=pl.BlockSpec((R, U), lambda i: (i, 0)),
        out_shape=jax.ShapeDtypeStruct((N, U), jnp.float32),
    )(pf, pw, embedding)


def kernel(coord_features, indices, weights, embedding):
    wcoord = coord_features * weights[:, None]
    idx2d = indices.reshape(indices.shape[0] // Q, Q)
    pf, pw = _sc_segment_accumulate(wcoord, idx2d, weights)
    return _tc_finish(pf, pw, embedding)
